# Initial kernel scaffold; baseline (speedup 1.0000x reference)
import jax
import jax.numpy as jnp
from jax.experimental import pallas as pl


def kernel(x, in_proj_w, conv_w, conv_b, x_proj_w, dt_proj_w, dt_proj_b, log_A, D_skip, out_proj_w):
    raise NotImplementedError("write your pallas kernel here")


# speedup vs baseline: 23.3166x; 23.3166x over previous
import jax
import jax.numpy as jnp
from jax.experimental import pallas as pl
from jax.experimental.pallas import tpu as pltpu

D_MODEL = 768
D_STATE = 16
D_CONV = 4
D_INNER = 1536
DT_RANK = 48
SEQ = 2048
T_CHUNK = 256
N_CHUNKS = SEQ // T_CHUNK


def _mamba_body(x_ref, w1t_ref, wconv_ref, cb_ref, wxt_ref, wdt_ref, dtb_ref,
                logAT_ref, dsk_ref, wot_ref, o_ref,
                delta_ref, u_ref, xbr_ref, z_ref, bc_ref, y_ref, h_ref, cc_ref):
    i = pl.program_id(1)

    @pl.when(i == 0)
    def _():
        h_ref[...] = jnp.zeros_like(h_ref)
        cc_ref[...] = jnp.zeros_like(cc_ref)

    T = T_CHUNK
    xc = x_ref[0]
    xz = jnp.dot(xc, w1t_ref[...], preferred_element_type=jnp.float32)
    xb = xz[:, :D_INNER]
    z_ref[...] = xz[:, D_INNER:]

    prev3 = cc_ref[5:8, :]
    ext = jnp.concatenate([prev3, xb], axis=0)
    conv = (wconv_ref[0:1, :] * ext[0:T, :]
            + wconv_ref[1:2, :] * ext[1:T + 1, :]
            + wconv_ref[2:3, :] * ext[2:T + 2, :]
            + wconv_ref[3:4, :] * ext[3:T + 3, :]) + cb_ref[...]
    cc_ref[5:8, :] = xb[T - 3:T, :]
    xbr = conv * jax.nn.sigmoid(conv)
    xbr_ref[...] = xbr

    dbc = jnp.dot(xbr, wxt_ref[...], preferred_element_type=jnp.float32)
    bc_ref[...] = dbc[:, DT_RANK:DT_RANK + 2 * D_STATE]
    delta = jax.nn.softplus(
        jnp.dot(dbc[:, :DT_RANK], wdt_ref[...],
                preferred_element_type=jnp.float32) + dtb_ref[...])
    delta_ref[...] = delta
    u_ref[...] = delta * xbr

    aneg = -jnp.exp(logAT_ref[...])

    def slab(s, h):
        base = pl.multiple_of(s * 8, 8)
        d8 = delta_ref[pl.ds(base, 8), :]
        u8 = u_ref[pl.ds(base, 8), :]
        bc8 = bc_ref[pl.ds(base, 8), :]
        bt = bc8[:, 0:D_STATE].T
        ct = bc8[:, D_STATE:2 * D_STATE].T
        a8 = jnp.exp(d8[:, None, :] * aneg[None, :, :])
        ys = []
        for r in range(8):
            bx = bt[:, r:r + 1] * u8[r:r + 1, :]
            h = a8[r] * h + bx
            ys.append(jnp.sum(ct[:, r:r + 1] * h, axis=0, keepdims=True))
        y_ref[pl.ds(base, 8), :] = jnp.concatenate(ys, axis=0)
        return h

    h = jax.lax.fori_loop(0, T // 8, slab, h_ref[...])
    h_ref[...] = h

    zv = z_ref[...]
    yg = (y_ref[...] + dsk_ref[...] * xbr_ref[...]) * (zv * jax.nn.sigmoid(zv))
    o_ref[0] = jnp.dot(yg, wot_ref[...], preferred_element_type=jnp.float32)


def kernel(x, in_proj_w, conv_w, conv_b, x_proj_w, dt_proj_w, dt_proj_b,
           log_A, D_skip, out_proj_w, interpret=False):
    B, S, D = x.shape
    w1t = in_proj_w.T
    wxt = x_proj_w.T
    wdt = dt_proj_w.T
    wot = out_proj_w.T
    wconv = conv_w[:, 0, :].T
    cb = conv_b[None, :]
    dtb = dt_proj_b[None, :]
    logAT = log_A.T
    dsk = D_skip[None, :]

    full = lambda shape: pl.BlockSpec(shape, lambda b, i: (0,) * len(shape))
    grid = (B, N_CHUNKS)
    return pl.pallas_call(
        _mamba_body,
        grid=grid,
        in_specs=[
            pl.BlockSpec((1, T_CHUNK, D), lambda b, i: (b, i, 0)),
            full((D, 2 * D_INNER)),
            full((D_CONV, D_INNER)),
            full((1, D_INNER)),
            full((D_INNER, DT_RANK + 2 * D_STATE)),
            full((DT_RANK, D_INNER)),
            full((1, D_INNER)),
            full((D_STATE, D_INNER)),
            full((1, D_INNER)),
            full((D_INNER, D)),
        ],
        out_specs=pl.BlockSpec((1, T_CHUNK, D), lambda b, i: (b, i, 0)),
        out_shape=jax.ShapeDtypeStruct((B, S, D), jnp.float32),
        scratch_shapes=[
            pltpu.VMEM((T_CHUNK, D_INNER), jnp.float32),
            pltpu.VMEM((T_CHUNK, D_INNER), jnp.float32),
            pltpu.VMEM((T_CHUNK, D_INNER), jnp.float32),
            pltpu.VMEM((T_CHUNK, D_INNER), jnp.float32),
            pltpu.VMEM((T_CHUNK, 2 * D_STATE), jnp.float32),
            pltpu.VMEM((T_CHUNK, D_INNER), jnp.float32),
            pltpu.VMEM((D_STATE, D_INNER), jnp.float32),
            pltpu.VMEM((8, D_INNER), jnp.float32),
        ],
        compiler_params=pltpu.CompilerParams(
            dimension_semantics=("parallel", "arbitrary"),
            vmem_limit_bytes=56 * 1024 * 1024,
        ),
        name="mamba_ssm_fused",
        interpret=interpret,
    )(x, w1t, wconv, cb, wxt, wdt, dtb, logAT, dsk, wot)
